# baseline (speedup 1.0000x reference)
import functools

import numpy as np

import jax
import jax.numpy as jnp
from jax import lax
from jax.experimental import pallas as pl
from jax.experimental.pallas import tpu as pltpu
from jax.experimental.pallas import tpu_sc as plsc

P = 2147483647
N_ROBE = 64000
FEAT = 64
A1, B1 = 1664525, 1013904223
A2, A3, B2 = 22695477, 1103515245, 12345
A1H, A1L = A1 >> 8, A1 & 0xFF
A2H, A2L = A2 >> 12, A2 & 0xFFF
CD = [(A3 * d + B2) % P for d in range(FEAT)]

NCORE, NSUB = 2, 16
NW = NCORE * NSUB
NIDX = 4096 * 26
NI = NIDX // NW
CI = 256
NCHUNK = NI // CI
ROW_W = FEAT + 1
CHUNK_P = CI * ROW_W


def _addmod(a, b):
    d = a - (P - b)
    return jnp.where(d < 0, a + b, d)


def _mod_robe(r):
    for _ in range(3):
        r = (r >> 16) * 1536 + (r & 0xFFFF)
    d = r - N_ROBE
    return jnp.where(d < 0, r, d)


def _hashes(x):
    xh, xl = x >> 16, x & 0xFFFF
    t1 = A1 * xh
    t1m = ((t1 & 0x7FFF) << 16) + (t1 >> 15)
    t2 = A1H * xl
    t2m = ((t2 & 0x7FFFFF) << 8) + (t2 >> 23)
    t3 = A1L * xl
    h = _mod_robe(_addmod(_addmod(t1m, t2m), _addmod(t3, jnp.int32(B1))))
    u1 = A2 * xh
    u1m = ((u1 & 0x7FFF) << 16) + (u1 >> 15)
    u2 = A2H * xl
    u2m = ((u2 & 0x7FFFF) << 12) + (u2 >> 19)
    u3 = A2L * xl
    a2 = _addmod(_addmod(u1m, u2m), u3)
    return h, a2


def _i32(v):
    return jnp.int32(v)


def _body(robe_hbm, idx_hbm, out_hbm,
          robe_v, idx_v, h_v, a2_v, thr_s, par_s, stage_a, stage_b, sem0, sem1):
    wid = (lax.axis_index("s").astype(jnp.int32) * _i32(NCORE)
           + lax.axis_index("c").astype(jnp.int32))
    ibase = wid * _i32(NI)
    pltpu.sync_copy(robe_hbm, robe_v.at[pl.ds(_i32(0), N_ROBE)])
    pltpu.sync_copy(robe_hbm.at[pl.ds(_i32(0), FEAT)],
                    robe_v.at[pl.ds(_i32(N_ROBE), FEAT)])
    pltpu.sync_copy(idx_hbm.at[pl.ds(ibase, NI)], idx_v)

    def fill_tab(d, c):
        thr_s[d] = _i32(P - 1) - c
        par_s[d] = (c & _i32(1)) << _i32(31)
        t = c - _i32(P - A3)
        return jnp.where(t < 0, c + _i32(A3), t)

    lax.fori_loop(_i32(0), _i32(FEAT), fill_tab, _i32(B2))
    lane = lax.iota(jnp.int32, 16)

    def compute_chunk(ci, buf):
        @plsc.parallel_loop(_i32(0), _i32(CI // 16), step=_i32(1), unroll=2)
        def hpass(g):
            x = idx_v[pl.ds(ci * _i32(CI) + g * _i32(16), 16)]
            h, a2 = _hashes(x)
            h_v[pl.ds(g * _i32(16), 16)] = h
            a2_v[pl.ds(g * _i32(16), 16)] = a2

        def group(g, _):
            h = h_v[pl.ds(g * _i32(16), 16)]
            a2 = a2_v[pl.ds(g * _i32(16), 16)]
            px = a2 & 1
            f0 = (px ^ 1) << 31
            f1 = f0 ^ jnp.int32(-2**31)
            nvec = lane + g * _i32(16)

            @plsc.parallel_loop(_i32(0), _i32(FEAT), step=_i32(1), unroll=16)
            def dloop(d):
                thr = thr_s[d]
                par = par_s[d]
                gv = plsc.load_gather(robe_v, [h + d])
                sel = jnp.where(a2 > thr, f1, f0)
                outw = plsc.bitcast(gv, jnp.int32) ^ sel ^ par
                plsc.store_scatter(buf, [nvec, jnp.broadcast_to(d, (16,))],
                                   plsc.bitcast(outw, jnp.float32))

            return _i32(0)

        lax.fori_loop(_i32(0), _i32(CI // 16), group, _i32(0))

    def fire(ci, buf, sem):
        row0 = wid * _i32(NI) + ci * _i32(CI)
        pltpu.async_copy(buf.at[:, pl.ds(_i32(0), FEAT)],
                         out_hbm.at[pl.ds(row0, CI), :], sem)

    def drain(buf, sem):
        pltpu.make_async_copy(buf.at[:, pl.ds(_i32(0), FEAT)],
                              out_hbm.at[pl.ds(_i32(0), CI), :], sem).wait()

    def chunk_pair(cp, _):
        ci0 = cp * _i32(2)

        @pl.when(cp > 0)
        def _():
            drain(stage_a, sem0)

        compute_chunk(ci0, stage_a)
        fire(ci0, stage_a, sem0)

        @pl.when(cp > 0)
        def _():
            drain(stage_b, sem1)

        compute_chunk(ci0 + _i32(1), stage_b)
        fire(ci0 + _i32(1), stage_b, sem1)
        return _i32(0)

    lax.fori_loop(_i32(0), _i32(NCHUNK // 2), chunk_pair, _i32(0))
    drain(stage_a, sem0)
    compute_chunk(_i32(NCHUNK - 1), stage_a)
    fire(_i32(NCHUNK - 1), stage_a, sem0)
    drain(stage_a, sem0)
    drain(stage_b, sem1)


_sc_call = pl.kernel(
    _body,
    out_type=jax.ShapeDtypeStruct((NIDX, FEAT), jnp.float32),
    mesh=plsc.VectorSubcoreMesh(
        core_axis_name="c", subcore_axis_name="s",
        num_cores=NCORE, num_subcores=NSUB),
    scratch_types=[
        pltpu.VMEM((N_ROBE + FEAT,), jnp.float32),
        pltpu.VMEM((NI,), jnp.int32),
        pltpu.VMEM((CI,), jnp.int32),
        pltpu.VMEM((CI,), jnp.int32),
        pltpu.SMEM((FEAT,), jnp.int32),
        pltpu.SMEM((FEAT,), jnp.int32),
        pltpu.VMEM((CI, ROW_W), jnp.float32),
        pltpu.VMEM((CI, ROW_W), jnp.float32),
        pltpu.SemaphoreType.DMA,
        pltpu.SemaphoreType.DMA,
    ],
    compiler_params=pltpu.CompilerParams(
        needs_layout_passes=False, use_tc_tiling_on_sc=False),
)


def kernel(robe_array, idx):
    idx32 = idx.reshape(-1).astype(jnp.int32)
    out = _sc_call(robe_array, idx32)
    return out.reshape(idx.shape[0], idx.shape[1], FEAT)


# speedup vs baseline: 1.0087x; 1.0087x over previous
import functools

import numpy as np

import jax
import jax.numpy as jnp
from jax import lax
from jax.experimental import pallas as pl
from jax.experimental.pallas import tpu as pltpu
from jax.experimental.pallas import tpu_sc as plsc

P = 2147483647
N_ROBE = 64000
FEAT = 64
A1, B1 = 1664525, 1013904223
A2, A3, B2 = 22695477, 1103515245, 12345
A1H, A1L = A1 >> 8, A1 & 0xFF
A2H, A2L = A2 >> 12, A2 & 0xFFF
CD = [(A3 * d + B2) % P for d in range(FEAT)]

NCORE, NSUB = 2, 16
NW = NCORE * NSUB
NIDX = 4096 * 26
NI = NIDX // NW
CI = 256
NCHUNK = NI // CI
ROW_W = FEAT + 1
CHUNK_P = CI * ROW_W


def _addmod(a, b):
    d = a - (P - b)
    return jnp.where(d < 0, a + b, d)


def _mod_robe(r):
    for _ in range(3):
        r = (r >> 16) * 1536 + (r & 0xFFFF)
    d = r - N_ROBE
    return jnp.where(d < 0, r, d)


def _hashes(x):
    xh, xl = x >> 16, x & 0xFFFF
    t1 = A1 * xh
    t1m = ((t1 & 0x7FFF) << 16) + (t1 >> 15)
    t2 = A1H * xl
    t2m = ((t2 & 0x7FFFFF) << 8) + (t2 >> 23)
    t3 = A1L * xl
    h = _mod_robe(_addmod(_addmod(t1m, t2m), _addmod(t3, jnp.int32(B1))))
    u1 = A2 * xh
    u1m = ((u1 & 0x7FFF) << 16) + (u1 >> 15)
    u2 = A2H * xl
    u2m = ((u2 & 0x7FFFF) << 12) + (u2 >> 19)
    u3 = A2L * xl
    a2 = _addmod(_addmod(u1m, u2m), u3)
    return h, a2


def _i32(v):
    return jnp.int32(v)


def _body(robe_hbm, idx_hbm, out_hbm,
          robe_v, idx_v, h_v, a2_v, thr_s, par_s, stage_a, stage_b, sem0, sem1):
    wid = (lax.axis_index("s").astype(jnp.int32) * _i32(NCORE)
           + lax.axis_index("c").astype(jnp.int32))
    ibase = wid * _i32(NI)
    pltpu.sync_copy(robe_hbm, robe_v.at[pl.ds(_i32(0), N_ROBE)])
    pltpu.sync_copy(robe_hbm.at[pl.ds(_i32(0), FEAT)],
                    robe_v.at[pl.ds(_i32(N_ROBE), FEAT)])
    pltpu.sync_copy(idx_hbm.at[pl.ds(ibase, NI)], idx_v)

    def fill_tab(d, c):
        thr_s[d] = _i32(P - 1) - c
        par_s[d] = (c & _i32(1)) << _i32(31)
        t = c - _i32(P - A3)
        return jnp.where(t < 0, c + _i32(A3), t)

    lax.fori_loop(_i32(0), _i32(FEAT), fill_tab, _i32(B2))
    lane = lax.iota(jnp.int32, 16)

    def compute_chunk(ci, buf):
        @plsc.parallel_loop(_i32(0), _i32(CI // 16), step=_i32(1), unroll=2)
        def hpass(g):
            x = idx_v[pl.ds(ci * _i32(CI) + g * _i32(16), 16)]
            h, a2 = _hashes(x)
            h_v[pl.ds(g * _i32(16), 16)] = h
            a2_v[pl.ds(g * _i32(16), 16)] = a2

        def group(g, _):
            h = h_v[pl.ds(g * _i32(16), 16)]
            a2 = a2_v[pl.ds(g * _i32(16), 16)]
            px = a2 & 1
            f0 = (px ^ 1) << 31
            f1 = f0 ^ jnp.int32(-2**31)
            nvec = lane + g * _i32(16)

            @plsc.parallel_loop(_i32(0), _i32(FEAT), step=_i32(1), unroll=8)
            def dloop(d):
                thr = thr_s[d]
                par = par_s[d]
                gv = plsc.load_gather(robe_v, [h + d])
                sel = jnp.where(a2 > thr, f1, f0)
                outw = plsc.bitcast(gv, jnp.int32) ^ sel ^ par
                plsc.store_scatter(buf, [nvec, jnp.broadcast_to(d, (16,))],
                                   plsc.bitcast(outw, jnp.float32))

            return _i32(0)

        lax.fori_loop(_i32(0), _i32(CI // 16), group, _i32(0))

    def fire(ci, buf, sem):
        row0 = wid * _i32(NI) + ci * _i32(CI)
        pltpu.async_copy(buf.at[:, pl.ds(_i32(0), FEAT)],
                         out_hbm.at[pl.ds(row0, CI), :], sem)

    def drain(buf, sem):
        pltpu.make_async_copy(buf.at[:, pl.ds(_i32(0), FEAT)],
                              out_hbm.at[pl.ds(_i32(0), CI), :], sem).wait()

    def chunk_pair(cp, _):
        ci0 = cp * _i32(2)

        @pl.when(cp > 0)
        def _():
            drain(stage_a, sem0)

        compute_chunk(ci0, stage_a)
        fire(ci0, stage_a, sem0)

        @pl.when(cp > 0)
        def _():
            drain(stage_b, sem1)

        compute_chunk(ci0 + _i32(1), stage_b)
        fire(ci0 + _i32(1), stage_b, sem1)
        return _i32(0)

    lax.fori_loop(_i32(0), _i32(NCHUNK // 2), chunk_pair, _i32(0))
    drain(stage_a, sem0)
    compute_chunk(_i32(NCHUNK - 1), stage_a)
    fire(_i32(NCHUNK - 1), stage_a, sem0)
    drain(stage_a, sem0)
    drain(stage_b, sem1)


_sc_call = pl.kernel(
    _body,
    out_type=jax.ShapeDtypeStruct((NIDX, FEAT), jnp.float32),
    mesh=plsc.VectorSubcoreMesh(
        core_axis_name="c", subcore_axis_name="s",
        num_cores=NCORE, num_subcores=NSUB),
    scratch_types=[
        pltpu.VMEM((N_ROBE + FEAT,), jnp.float32),
        pltpu.VMEM((NI,), jnp.int32),
        pltpu.VMEM((CI,), jnp.int32),
        pltpu.VMEM((CI,), jnp.int32),
        pltpu.SMEM((FEAT,), jnp.int32),
        pltpu.SMEM((FEAT,), jnp.int32),
        pltpu.VMEM((CI, ROW_W), jnp.float32),
        pltpu.VMEM((CI, ROW_W), jnp.float32),
        pltpu.SemaphoreType.DMA,
        pltpu.SemaphoreType.DMA,
    ],
    compiler_params=pltpu.CompilerParams(
        needs_layout_passes=False, use_tc_tiling_on_sc=False),
)


def kernel(robe_array, idx):
    idx32 = idx.reshape(-1).astype(jnp.int32)
    out = _sc_call(robe_array, idx32)
    return out.reshape(idx.shape[0], idx.shape[1], FEAT)
